# baseline (speedup 1.0000x reference)
import functools

import jax
import jax.numpy as jnp
from jax import lax
from jax.experimental import pallas as pl
from jax.experimental.pallas import tpu as pltpu
from jax.experimental.pallas import tpu_sc as plsc

B = 4096
T = 50
D = 16
NW = 32
NWIDE = 26
NDEEP = 26
BPW = B // NW
SEQR = T * BPW
SCH = SEQR // 2
PCK = 200


def _sc_gather(total, din, wide_t, deep_t, seq1, seq3, cu_s, tgt_i, oth_i,
               wide_i, deep_i):
    f32, i32 = jnp.float32, jnp.int32
    mesh = plsc.VectorSubcoreMesh(core_axis_name="c", subcore_axis_name="s")

    @functools.partial(
        pl.kernel,
        out_type=(
            jax.ShapeDtypeStruct((NW, SEQR // 8, 128), f32),
            jax.ShapeDtypeStruct((NW, SEQR // 8, 128), f32),
            jax.ShapeDtypeStruct((NW, NWIDE * BPW, D), f32),
            jax.ShapeDtypeStruct((NW, NDEEP * BPW, D), f32),
            jax.ShapeDtypeStruct((NW, 2 * BPW, D), f32),
            jax.ShapeDtypeStruct((NW, BPW, D), f32),
            jax.ShapeDtypeStruct((NW, 2 * BPW // 8, 128), f32),
        ),
        mesh=mesh,
        compiler_params=pltpu.CompilerParams(use_tc_tiling_on_sc=False),
        scratch_types=[
            pltpu.VMEM((BPW,), i32),
            pltpu.VMEM((2, SCH), i32),
            pltpu.VMEM((2, SCH), i32),
            pltpu.VMEM((NWIDE * BPW,), i32),
            pltpu.VMEM((2 * BPW,), i32),
            pltpu.VMEM((BPW,), i32),
            pltpu.VMEM((NWIDE * BPW, D), f32),
            pltpu.VMEM((PCK, 128), f32),
            pltpu.SemaphoreType.DMA,
            pltpu.SemaphoreType.DMA,
        ],
    )
    def k(din_h, wide_th, deep_th, s1_h, s3_h, cu_h, ti_h, oi_h, wi_h, di_h,
          s1_o, s3_o, wide_o, deep_o, tgt_o, oth_o, tgtp_o,
          cu_v, pos_v, ids_v, wd_v, t_v, o_v, rows_v, pck_v, sem, sem2):
        wid = lax.axis_index("s") * 2 + lax.axis_index("c")

        pltpu.sync_copy(cu_h.at[pl.ds(wid * BPW, BPW)], cu_v)
        pltpu.async_copy(ti_h.at[pl.ds(wid * 2 * BPW, 2 * BPW)], t_v, sem2)
        pltpu.async_copy(oi_h.at[pl.ds(wid * BPW, BPW)], o_v, sem2)
        for i in range(BPW // 16):
            sv = cu_v[pl.ds(16 * i, 16)]
            for t in range(T):
                n = t * BPW + 16 * i
                pos_v[n // SCH, pl.ds(n % SCH, 16)] = jnp.minimum(
                    sv + t, total - 1)
        for c in range(2):
            pltpu.async_copy(s1_h.at[pos_v.at[c]], ids_v.at[c], sem)
        pltpu.make_async_copy(s1_h.at[pos_v.at[1]], ids_v.at[1], sem).wait()
        pltpu.make_async_copy(s1_h.at[pos_v.at[0]], ids_v.at[0], sem).wait()
        for si, out_h in ((0, s1_o), (1, s3_o)):
            for c in range(2):
                pltpu.async_copy(din_h.at[ids_v.at[c]],
                                 rows_v.at[pl.ds(0, SCH)], sem).wait()
                if si == 0:
                    pltpu.async_copy(s3_h.at[pos_v.at[c]], ids_v.at[c], sem2)

                for s in range(SCH // (8 * PCK)):
                    def body(lp, carry, _s=s):
                        for kk in range(8):
                            pck_v[lp, pl.ds(16 * kk, 16)] = rows_v[
                                _s * 8 * PCK + lp * 8 + kk, :]
                        return carry

                    lax.fori_loop(0, PCK, body, 0)
                    pltpu.sync_copy(
                        pck_v,
                        out_h.at[wid,
                                 pl.ds(c * (SCH // 8) + s * PCK, PCK)])
            if si == 0:
                for c in range(2):
                    pltpu.make_async_copy(
                        s3_h.at[pos_v.at[c]], ids_v.at[c], sem2).wait()
        pltpu.sync_copy(wi_h.at[pl.ds(wid * NWIDE * BPW, NWIDE * BPW)], wd_v)
        pltpu.async_copy(wide_th.at[wd_v], rows_v, sem).wait()
        pltpu.sync_copy(rows_v, wide_o.at[wid])
        pltpu.sync_copy(di_h.at[pl.ds(wid * NDEEP * BPW, NDEEP * BPW)], wd_v)
        pltpu.async_copy(deep_th.at[wd_v], rows_v, sem).wait()
        pltpu.sync_copy(rows_v, deep_o.at[wid])
        pltpu.make_async_copy(
            ti_h.at[pl.ds(wid * 2 * BPW, 2 * BPW)], t_v, sem2).wait()
        pltpu.async_copy(din_h.at[t_v],
                         rows_v.at[pl.ds(0, 2 * BPW)], sem).wait()
        pltpu.sync_copy(rows_v.at[pl.ds(0, 2 * BPW)], tgt_o.at[wid])

        def tbody(lp, carry):
            for kk in range(8):
                pck_v[lp, pl.ds(16 * kk, 16)] = rows_v[lp * 8 + kk, :]
            return carry

        lax.fori_loop(0, 2 * BPW // 8, tbody, 0)
        pltpu.sync_copy(pck_v.at[pl.ds(0, 2 * BPW // 8)], tgtp_o.at[wid])
        pltpu.make_async_copy(
            oi_h.at[pl.ds(wid * BPW, BPW)], o_v, sem2).wait()
        pltpu.async_copy(din_h.at[o_v], rows_v.at[pl.ds(0, BPW)], sem).wait()
        pltpu.sync_copy(rows_v.at[pl.ds(0, BPW)], oth_o.at[wid])

    return k(din, wide_t, deep_t, seq1, seq3, cu_s, tgt_i, oth_i, wide_i, deep_i)


def _dice(x, a):
    p = jax.nn.sigmoid(x)
    return p * x + (1.0 - p) * a * x


def _tc_body(s1_r, s3_r, tgtp_r, tgtf_r, oth_r, wide_r, deep_r, len_r,
             aWq1_r, aWq3_r, aWs1_r, aWs3_r, aWc1_r, aWc3_r, ab1_r, aa1_r,
             aW2_r, ab2_r, aa2_r, aM3_r, ab3_r,
             mW1_r, mb1_r, ma1_r, mW2_r, mb2_r, ma2_r, mW3_r, mb3_r,
             lw_r, lb_r, dW1_r, db1_r, dW2_r, db2_r, dW3_r, db3_r, out_r):
    BB = BPW
    s1p = s1_r[0].reshape(T, 16, 128)
    s3p = s3_r[0].reshape(T, 16, 128)
    qq = tgtp_r[0].reshape(2, 16, 128)
    q1p = qq[0]
    q3p = qq[1]
    TX = T * 16
    s1f = s1p.reshape(TX, 128)
    s3f = s3p.reshape(TX, 128)
    qs1 = (q1p[None] * s1p).reshape(TX, 128)
    qs3 = (q3p[None] * s3p).reshape(TX, 128)
    term = (s1f @ aWs1_r[...] + s3f @ aWs3_r[...]
            + qs1 @ aWc1_r[...] + qs3 @ aWc3_r[...])
    tq = q1p @ aWq1_r[...] + q3p @ aWq3_r[...]
    h = term.reshape(T, 16, 128) + tq[None] + ab1_r[0][None, None]
    h = _dice(h, aa1_r[0])
    h = (h.reshape(TX, 128) @ aW2_r[...]).reshape(T, 16, 128) + ab2_r[0]
    h = _dice(h, aa2_r[0])
    scores = (h.reshape(TX, 128) @ aM3_r[...]).reshape(T, 16, 128) + ab3_r[0, 0]
    lens = len_r[0]
    tiota = lax.broadcasted_iota(jnp.int32, (T, 16, 128), 0)
    scores = jnp.where(tiota < lens[None], scores, -1e9)
    m = jnp.max(scores, axis=0)
    e = jnp.exp(scores - m[None])
    w = e / jnp.sum(e, axis=0)[None]
    p1 = jnp.sum(w * s1p, axis=0).reshape(BB, D)
    p3 = jnp.sum(w * s3p, axis=0).reshape(BB, D)
    q = tgtf_r[0]
    oth = oth_r[0]
    mW1 = mW1_r[...]
    h2 = (oth @ mW1[0:D] + p1 @ mW1[D:2 * D] + p3 @ mW1[2 * D:3 * D]
          + q @ mW1[3 * D:5 * D]) + mb1_r[0]
    h2 = _dice(h2, ma1_r[0])
    h2 = _dice(h2 @ mW2_r[...] + mb2_r[0], ma2_r[0])
    dout = h2 @ mW3_r[...] + mb3_r[0, 0]
    dout = dout + wide_r[0] @ lw_r[...] + lb_r[0, 0]
    hd = jnp.maximum(deep_r[0] @ dW1_r[...] + db1_r[0], 0.0)
    hd = jnp.maximum(hd @ dW2_r[...] + db2_r[0], 0.0)
    dout = dout + hd @ dW3_r[...] + db3_r[0, 0]
    out_r[0] = jax.nn.sigmoid(dout)


def _row2(x):
    return x.reshape(1, -1)


def kernel(params, seq_ids_1, seq_ids_3, cu_seqlens, target_ids, other_ids,
           wide_ids, deep_ids):
    p = params
    f32 = jnp.float32
    total = seq_ids_1.shape[0]
    cu = cu_seqlens.astype(jnp.int32)
    len_exp = jnp.repeat((cu[1:] - cu[:-1]).reshape(NW, 16, 8), D, axis=-1)

    kron8 = lambda w: jnp.kron(jnp.eye(8, dtype=f32), w)
    W1 = p['att_W1']
    Wq = W1[0:2 * D] + W1[4 * D:6 * D]
    Ws = W1[2 * D:4 * D] - W1[4 * D:6 * D]
    Wc = W1[6 * D:8 * D]
    W2p = jnp.pad(p['att_W2'], ((0, 0), (0, 8)))
    W3pad = jnp.concatenate([p['att_W3'][:, 0], jnp.zeros((8,), f32)])
    M3 = kron8(W3pad[:, None] * jnp.ones((1, 16), f32))
    tile8 = lambda v: jnp.tile(v, 8).reshape(1, 128)
    ab1p, aa1p = tile8(p['att_b1']), tile8(p['att_a1'])
    ab2p = tile8(jnp.pad(p['att_b2'], (0, 8)))
    aa2p = tile8(jnp.pad(p['att_a2'], (0, 8)))

    full = lambda shape: pl.BlockSpec(shape, lambda i: (0,) * len(shape))
    grid_spec = pl.GridSpec(
        grid=(NW,),
        in_specs=[
            pl.BlockSpec((1, SEQR // 8, 128), lambda i: (i, 0, 0)),
            pl.BlockSpec((1, SEQR // 8, 128), lambda i: (i, 0, 0)),
            pl.BlockSpec((1, 2 * BPW // 8, 128), lambda i: (i, 0, 0)),
            pl.BlockSpec((1, BPW, 2 * D), lambda i: (i, 0, 0)),
            pl.BlockSpec((1, BPW, D), lambda i: (i, 0, 0)),
            pl.BlockSpec((1, BPW, NWIDE * D), lambda i: (i, 0, 0)),
            pl.BlockSpec((1, BPW, NDEEP * D), lambda i: (i, 0, 0)),
            pl.BlockSpec((1, 16, 128), lambda i: (i, 0, 0)),
            full((128, 128)), full((128, 128)), full((128, 128)),
            full((128, 128)), full((128, 128)), full((128, 128)),
            full((1, 128)), full((1, 128)),
            full((128, 128)), full((1, 128)), full((1, 128)),
            full((128, 128)), full((1, 1)),
            full((5 * D, 32)), full((1, 32)), full((1, 32)),
            full((32, 16)), full((1, 16)), full((1, 16)),
            full((16, 1)), full((1, 1)),
            full((NWIDE * D, 1)), full((1, 1)),
            full((NDEEP * D, 32)), full((1, 32)),
            full((32, 16)), full((1, 16)),
            full((16, 1)), full((1, 1)),
        ],
        out_specs=pl.BlockSpec((1, BPW, 1), lambda i: (i, 0, 0)),
    )
    tc_call = pl.pallas_call(
        _tc_body,
        grid_spec=grid_spec,
        out_shape=jax.ShapeDtypeStruct((NW, BPW, 1), jnp.float32),
        compiler_params=pltpu.CompilerParams(
            dimension_semantics=("parallel",)),
    )
    weights = (
        kron8(Wq[:D]), kron8(Wq[D:]), kron8(Ws[:D]), kron8(Ws[D:]),
        kron8(Wc[:D]), kron8(Wc[D:]),
        ab1p, aa1p,
        kron8(W2p), ab2p, aa2p,
        M3, _row2(p['att_b3']),
        p['mlp_W1'], _row2(p['mlp_b1']), _row2(p['mlp_a1']),
        p['mlp_W2'], _row2(p['mlp_b2']), _row2(p['mlp_a2']),
        p['mlp_W3'], _row2(p['mlp_b3']),
        p['lr_w'], _row2(p['lr_b']),
        p['deep_W1'], _row2(p['deep_b1']),
        p['deep_W2'], _row2(p['deep_b2']),
        p['deep_W3'], _row2(p['deep_b3']),
    )

    rowmajor = lambda t: lax.optimization_barrier(
        t.reshape(-1)).reshape(t.shape)
    din_t = rowmajor(p['din_table'])
    wide_t = rowmajor(p['wide_table'])
    deep_t = rowmajor(p['deep_table'])

    tgt_km = target_ids.reshape(NW, BPW, 2).transpose(0, 2, 1).reshape(-1)

    s1_g, s3_g, wide_g, deep_g, tgt_g, oth_g, tgtp_g = _sc_gather(
        total, din_t, wide_t, deep_t,
        seq_ids_1, seq_ids_3, cu[:B],
        tgt_km,
        other_ids.reshape(-1),
        wide_ids.reshape(-1),
        deep_ids.reshape(-1))
    tgt_f = (tgt_g.reshape(NW, 2, BPW, D)
             .transpose(0, 2, 1, 3).reshape(NW, BPW, 2 * D))
    out = tc_call(
        s1_g, s3_g, tgtp_g, tgt_f,
        oth_g.reshape(NW, BPW, D),
        wide_g.reshape(NW, BPW, NWIDE * D),
        deep_g.reshape(NW, BPW, NDEEP * D),
        len_exp,
        *weights)
    return out.reshape(B, 1)
